# baseline (speedup 1.0000x reference)
import jax
import jax.numpy as jnp
import numpy as np
from jax.experimental import pallas as pl

_NEW_T = np.arange(64, dtype=np.float32) / 64.0
_ROWS_PER_BLOCK = 256


def _interp_body(x_ref, o_ref):
    X = x_ref[...]
    R = X.shape[0]
    x = X[:, :64]
    y = X[:, 64:]
    t = jax.lax.broadcasted_iota(jnp.int32, (1, 64), 1).astype(jnp.float32) * (1.0 / 64.0)
    ones = jnp.ones((R, 1), jnp.float32)
    xf = jnp.concatenate([-0.01 * ones, x, 1.01 * ones], axis=1)
    yf = jnp.concatenate([0.0 * ones, y, 0.0 * ones], axis=1)
    m = (yf[:, 1:] - yf[:, :-1]) / (xf[:, 1:] - xf[:, :-1])
    b = yf[:, :65] - m * xf[:, :65]
    c = jnp.zeros((R, 64), jnp.float32)
    for j in range(64):
        c = c + (x[:, j:j + 1] <= t).astype(jnp.float32)
    acc = jnp.zeros((R, 64), jnp.float32)
    for k in range(65):
        term = m[:, k:k + 1] * t + b[:, k:k + 1]
        acc = jnp.where(c == k, term, acc)
    o_ref[...] = acc


def kernel(X):
    B = X.shape[0]
    R = _ROWS_PER_BLOCK
    return pl.pallas_call(
        _interp_body,
        grid=(B // R,),
        in_specs=[pl.BlockSpec((R, 128), lambda i: (i, 0))],
        out_specs=pl.BlockSpec((R, 64), lambda i: (i, 0)),
        out_shape=jax.ShapeDtypeStruct((B, 64), jnp.float32),
    )(X)


# speedup vs baseline: 3.6100x; 3.6100x over previous
import functools

import jax
import jax.numpy as jnp
from jax import lax
from jax.experimental import pallas as pl
from jax.experimental.pallas import tpu as pltpu
from jax.experimental.pallas import tpu_sc as plsc

_B = 131072
_NW = 32
_ROWS_PER_W = _B // _NW
_CHUNK = 256
_PASS = 16
_NPASS = _CHUNK // _PASS
_NCHUNK = _ROWS_PER_W // _CHUNK
_HB = 80
_MB = 66


def _sc_body(x_hbm, out_hbm, xv, ov, hist, mv, bv):
    nc = 2
    wid = lax.axis_index("s") * nc + lax.axis_index("c")
    iota = lax.iota(jnp.int32, 16)
    ones_i = jnp.ones((16,), jnp.int32)
    zeros_i = jnp.zeros((16,), jnp.int32)
    hbase = iota * _HB
    mbase = iota * _MB

    def chunk_body(ci):
        base = wid * _ROWS_PER_W + ci * _CHUNK
        pltpu.sync_copy(x_hbm.at[pl.ds(base * 128, _CHUNK * 128)], xv)

        def pass_body(pi):
            rowb = (iota + pi * _PASS) * 128
            outb = (iota + pi * _PASS) * 64

            for z in range(_PASS * _HB // 16):
                hist[pl.ds(z * 16, 16)] = zeros_i

            x_lo = jnp.full((16,), -0.01, jnp.float32)
            y_lo = jnp.zeros((16,), jnp.float32)
            for k in range(65):
                if k < 64:
                    x_hi = plsc.load_gather(xv, [rowb + k])
                    y_hi = plsc.load_gather(xv, [rowb + (64 + k)])
                    s = x_hi * 64.0
                    ti = s.astype(jnp.int32)
                    u = ti + (ti.astype(jnp.float32) < s).astype(jnp.int32)
                    plsc.addupdate_scatter(hist, [hbase + u], ones_i)
                else:
                    x_hi = jnp.full((16,), 1.01, jnp.float32)
                    y_hi = jnp.zeros((16,), jnp.float32)
                m = (y_hi - y_lo) / (x_hi - x_lo)
                b = y_lo - m * x_lo
                plsc.store_scatter(mv, [mbase + k], m)
                plsc.store_scatter(bv, [mbase + k], b)
                x_lo = x_hi
                y_lo = y_hi

            cnt = mbase
            for p in range(64):
                cnt = cnt + plsc.load_gather(hist, [hbase + p])
                mg = plsc.load_gather(mv, [cnt])
                bg = plsc.load_gather(bv, [cnt])
                o = mg * (p / 64.0) + bg
                plsc.store_scatter(ov, [outb + p], o)

        pl.loop(0, _NPASS)(pass_body)
        pltpu.sync_copy(ov, out_hbm.at[pl.ds(base * 64, _CHUNK * 64)])

    pl.loop(0, _NCHUNK)(chunk_body)


def kernel(X):
    mesh = plsc.VectorSubcoreMesh(core_axis_name="c", subcore_axis_name="s")
    f = functools.partial(
        pl.kernel,
        mesh=mesh,
        compiler_params=pltpu.CompilerParams(needs_layout_passes=False),
        out_type=jax.ShapeDtypeStruct((_B * 64,), jnp.float32),
        scratch_types=[
            pltpu.VMEM((_CHUNK * 128,), jnp.float32),
            pltpu.VMEM((_CHUNK * 64,), jnp.float32),
            pltpu.VMEM((_PASS * _HB,), jnp.int32),
            pltpu.VMEM((_PASS * _MB,), jnp.float32),
            pltpu.VMEM((_PASS * _MB,), jnp.float32),
        ],
    )(_sc_body)
    out = f(X.reshape(_B * 128))
    return out.reshape(_B, 64)
